# baseline (speedup 1.0000x reference)
import jax
import jax.numpy as jnp
from jax.experimental import pallas as pl
from jax.experimental.pallas import tpu as pltpu

_TB = 256


def _gather_kernel(ids_ref, mask_ref, vocab_ref, out_ref):
    tb = out_ref.shape[0]
    for mi in range(tb):
        idx = ids_ref[mi, 0]
        m = mask_ref[mi, 0]
        out_ref[mi] = vocab_ref[idx] * m


def kernel(tokens, vocab, mask):
    assert tokens.ndim == 2
    V, D = vocab.shape
    d0, d1 = tokens.shape
    N = d0 * d1

    tb = _TB if N >= _TB else max(8, pl.cdiv(N, 8) * 8)
    n_pad = pl.cdiv(N, tb) * tb
    pad = n_pad - N

    ids = tokens.reshape(-1).astype(jnp.int32)
    m = mask.reshape(-1).astype(jnp.float32)
    if pad:
        ids = jnp.pad(ids, (0, pad))
        m = jnp.pad(m, (0, pad))
    ids = ids.reshape(n_pad, 1)
    m = m.reshape(n_pad, 1)
    assert D % 8 == 0
    lanes = D // 8
    vocab3 = vocab.reshape(V, 8, lanes)

    grid = n_pad // tb
    table_bytes = V * D * jnp.dtype(vocab.dtype).itemsize
    tile_bytes = tb * D * 4
    vmem_limit = int(min(64 * 1024 * 1024,
                         2 * table_bytes + 4 * tile_bytes + (4 << 20)))

    out = pl.pallas_call(
        _gather_kernel,
        out_shape=jax.ShapeDtypeStruct((n_pad, 8, lanes), vocab.dtype),
        grid=(grid,),
        in_specs=[
            pl.BlockSpec((tb, 1), lambda i: (i, 0), memory_space=pltpu.SMEM),
            pl.BlockSpec((tb, 1), lambda i: (i, 0), memory_space=pltpu.SMEM),
            pl.BlockSpec((V, 8, lanes), lambda i: (0, 0, 0)),
        ],
        out_specs=pl.BlockSpec((tb, 8, lanes), lambda i: (i, 0, 0)),
        compiler_params=pltpu.CompilerParams(
            dimension_semantics=("parallel",),
            vmem_limit_bytes=vmem_limit,
        ),
    )(ids, m, vocab3)

    return out[:N].reshape(d0, d1, D)


# speedup vs baseline: 3.0915x; 3.0915x over previous
import jax
import jax.numpy as jnp
from jax.experimental import pallas as pl
from jax.experimental.pallas import tpu as pltpu

_TB = 256


def _gather_kernel(ids_ref, mask_ref, vocab_ref, out_ref):
    tb = out_ref.shape[0]
    base = pl.program_id(0) * tb
    for mi in range(tb):
        idx = ids_ref[base + mi]
        m = mask_ref[base + mi]
        out_ref[mi] = vocab_ref[idx] * m


def kernel(tokens, vocab, mask):
    assert tokens.ndim == 2
    V, D = vocab.shape
    d0, d1 = tokens.shape
    N = d0 * d1

    tb = _TB if N >= _TB else max(8, pl.cdiv(N, 8) * 8)
    n_pad = pl.cdiv(N, tb) * tb
    pad = n_pad - N

    ids = tokens.reshape(-1).astype(jnp.int32)
    m = mask.reshape(-1).astype(jnp.float32)
    if pad:
        ids = jnp.pad(ids, (0, pad))
        m = jnp.pad(m, (0, pad))
    assert D % 8 == 0
    lanes = D // 8
    vocab3 = vocab.reshape(V, 8, lanes)

    grid = n_pad // tb
    table_bytes = V * D * jnp.dtype(vocab.dtype).itemsize
    tile_bytes = tb * D * 4
    vmem_limit = int(min(64 * 1024 * 1024,
                         2 * table_bytes + 4 * tile_bytes + (4 << 20)))

    out = pl.pallas_call(
        _gather_kernel,
        out_shape=jax.ShapeDtypeStruct((n_pad, 8, lanes), vocab.dtype),
        grid_spec=pltpu.PrefetchScalarGridSpec(
            num_scalar_prefetch=2,
            grid=(grid,),
            in_specs=[
                pl.BlockSpec((V, 8, lanes), lambda i, ids, mm: (0, 0, 0)),
            ],
            out_specs=pl.BlockSpec((tb, 8, lanes), lambda i, ids, mm: (i, 0, 0)),
        ),
        compiler_params=pltpu.CompilerParams(
            dimension_semantics=("parallel",),
            vmem_limit_bytes=vmem_limit,
        ),
    )(ids, m, vocab3)

    return out[:N].reshape(d0, d1, D)


# speedup vs baseline: 6.8630x; 2.2200x over previous
import jax
import jax.numpy as jnp
from jax.experimental import pallas as pl
from jax.experimental.pallas import tpu as pltpu


def _gather_kernel(ids_ref, mask_ref, vocab_ref, out_ref):
    tb = out_ref.shape[1]
    d = out_ref.shape[2]
    base = pl.program_id(0) * tb
    for k in range(tb // 8):
        rows = []
        for t in range(8):
            j = base + 8 * k + t
            rows.append(vocab_ref[ids_ref[j]] * mask_ref[j])
        chunk = jnp.stack(rows, axis=0)
        out_ref[0, pl.ds(8 * k, 8), :] = chunk.reshape(8, d)


def kernel(tokens, vocab, mask):
    assert tokens.ndim == 2
    V, D = vocab.shape
    d0, d1 = tokens.shape
    assert d1 % 8 == 0 and D % 8 == 0

    ids = tokens.reshape(-1).astype(jnp.int32)
    m = mask.reshape(-1).astype(jnp.float32)
    vocab3 = vocab.reshape(V, 8, D // 8)

    table_bytes = V * D * jnp.dtype(vocab.dtype).itemsize
    tile_bytes = d1 * D * 4
    vmem_limit = int(min(64 * 1024 * 1024,
                         2 * table_bytes + 4 * tile_bytes + (4 << 20)))

    out = pl.pallas_call(
        _gather_kernel,
        out_shape=jax.ShapeDtypeStruct((d0, d1, D), vocab.dtype),
        grid_spec=pltpu.PrefetchScalarGridSpec(
            num_scalar_prefetch=2,
            grid=(d0,),
            in_specs=[
                pl.BlockSpec((V, 8, D // 8), lambda i, ids, mm: (0, 0, 0)),
            ],
            out_specs=pl.BlockSpec((1, d1, D), lambda i, ids, mm: (i, 0, 0)),
        ),
        compiler_params=pltpu.CompilerParams(
            dimension_semantics=("parallel",),
            vmem_limit_bytes=vmem_limit,
        ),
    )(ids, m, vocab3)

    return out


# speedup vs baseline: 8.4840x; 1.2362x over previous
import jax
import jax.numpy as jnp
from jax.experimental import pallas as pl
from jax.experimental.pallas import tpu as pltpu


def _gather_kernel(ids_ref, mask_ref, vocab_ref, out_ref):
    br, tb, d = out_ref.shape
    base = pl.program_id(0) * br * tb
    for r in range(br):
        for k in range(tb // 8):
            rows = []
            for t in range(8):
                j = base + r * tb + 8 * k + t
                rows.append(vocab_ref[ids_ref[j]] * mask_ref[j])
            chunk = jnp.stack(rows, axis=0)
            out_ref[r, pl.ds(8 * k, 8), :] = chunk.reshape(8, d)


def kernel(tokens, vocab, mask):
    assert tokens.ndim == 2
    V, D = vocab.shape
    d0, d1 = tokens.shape
    assert d1 % 8 == 0 and D % 8 == 0

    ids = tokens.reshape(-1).astype(jnp.int32)
    m = mask.reshape(-1).astype(jnp.float32)
    vocab3 = vocab.reshape(V, 8, D // 8)

    br = 2 if d0 % 2 == 0 else 1
    table_bytes = V * D * jnp.dtype(vocab.dtype).itemsize
    tile_bytes = br * d1 * D * 4
    vmem_limit = int(min(64 * 1024 * 1024,
                         2 * table_bytes + 4 * tile_bytes + (4 << 20)))

    out = pl.pallas_call(
        _gather_kernel,
        out_shape=jax.ShapeDtypeStruct((d0, d1, D), vocab.dtype),
        grid_spec=pltpu.PrefetchScalarGridSpec(
            num_scalar_prefetch=2,
            grid=(d0 // br,),
            in_specs=[
                pl.BlockSpec((V, 8, D // 8), lambda i, ids, mm: (0, 0, 0)),
            ],
            out_specs=pl.BlockSpec((br, d1, D), lambda i, ids, mm: (i, 0, 0)),
        ),
        compiler_params=pltpu.CompilerParams(
            dimension_semantics=("parallel",),
            vmem_limit_bytes=vmem_limit,
        ),
    )(ids, m, vocab3)

    return out


# speedup vs baseline: 9.6857x; 1.1416x over previous
import jax
import jax.numpy as jnp
from jax.experimental import pallas as pl
from jax.experimental.pallas import tpu as pltpu


def _gather_kernel(ids_ref, mask_ref, vocab_ref, out_ref):
    br, tb, d = out_ref.shape
    base = pl.program_id(0) * br * tb
    for r in range(br):
        for k in range(tb // 8):
            rows = []
            for t in range(8):
                j = base + r * tb + 8 * k + t
                rows.append(vocab_ref[ids_ref[j]] * mask_ref[j])
            chunk = jnp.stack(rows, axis=0)
            out_ref[r, pl.ds(8 * k, 8), :] = chunk.reshape(8, d)


def kernel(tokens, vocab, mask):
    assert tokens.ndim == 2
    V, D = vocab.shape
    d0, d1 = tokens.shape
    assert d1 % 8 == 0 and D % 8 == 0

    ids = tokens.reshape(-1).astype(jnp.int32)
    m = mask.reshape(-1).astype(jnp.float32)
    vocab3 = vocab.reshape(V, 8, D // 8)

    br = 4 if d0 % 4 == 0 else (2 if d0 % 2 == 0 else 1)
    table_bytes = V * D * jnp.dtype(vocab.dtype).itemsize
    tile_bytes = br * d1 * D * 4
    vmem_limit = int(min(64 * 1024 * 1024,
                         2 * table_bytes + 4 * tile_bytes + (4 << 20)))

    out = pl.pallas_call(
        _gather_kernel,
        out_shape=jax.ShapeDtypeStruct((d0, d1, D), vocab.dtype),
        grid_spec=pltpu.PrefetchScalarGridSpec(
            num_scalar_prefetch=2,
            grid=(d0 // br,),
            in_specs=[
                pl.BlockSpec((V, 8, D // 8), lambda i, ids, mm: (0, 0, 0)),
            ],
            out_specs=pl.BlockSpec((br, d1, D), lambda i, ids, mm: (i, 0, 0)),
        ),
        compiler_params=pltpu.CompilerParams(
            dimension_semantics=("parallel",),
            vmem_limit_bytes=vmem_limit,
        ),
    )(ids, m, vocab3)

    return out


# speedup vs baseline: 10.3110x; 1.0646x over previous
import jax
import jax.numpy as jnp
from jax.experimental import pallas as pl
from jax.experimental.pallas import tpu as pltpu


def _gather_kernel(ids_ref, mask_ref, vocab_ref, out_ref):
    br, tb, d = out_ref.shape
    base = pl.program_id(0) * br * tb
    for r in range(br):
        for k in range(tb // 8):
            rows = []
            for t in range(8):
                j = base + r * tb + 8 * k + t
                rows.append(vocab_ref[ids_ref[j]] * mask_ref[j])
            chunk = jnp.stack(rows, axis=0)
            out_ref[r, pl.ds(8 * k, 8), :] = chunk.reshape(8, d)


def kernel(tokens, vocab, mask):
    assert tokens.ndim == 2
    V, D = vocab.shape
    d0, d1 = tokens.shape
    assert d1 % 8 == 0 and D % 8 == 0

    ids = tokens.reshape(-1).astype(jnp.int32)
    m = mask.reshape(-1).astype(jnp.float32)
    vocab3 = vocab.reshape(V, 8, D // 8)

    br = 1
    for cand in (8, 4, 2):
        if d0 % cand == 0:
            br = cand
            break
    table_bytes = V * D * jnp.dtype(vocab.dtype).itemsize
    tile_bytes = br * d1 * D * 4
    vmem_limit = int(min(64 * 1024 * 1024,
                         2 * table_bytes + 4 * tile_bytes + (4 << 20)))

    out = pl.pallas_call(
        _gather_kernel,
        out_shape=jax.ShapeDtypeStruct((d0, d1, D), vocab.dtype),
        grid_spec=pltpu.PrefetchScalarGridSpec(
            num_scalar_prefetch=2,
            grid=(d0 // br,),
            in_specs=[
                pl.BlockSpec((V, 8, D // 8), lambda i, ids, mm: (0, 0, 0)),
            ],
            out_specs=pl.BlockSpec((br, d1, D), lambda i, ids, mm: (i, 0, 0)),
        ),
        compiler_params=pltpu.CompilerParams(
            dimension_semantics=("parallel",),
            vmem_limit_bytes=vmem_limit,
        ),
    )(ids, m, vocab3)

    return out
